# baseline (speedup 1.0000x reference)
import numpy as np
import jax
import jax.numpy as jnp
from jax import lax
from jax.experimental import pallas as pl
from jax.experimental.pallas import tpu as pltpu


def _elu(v):
    return jnp.where(v > 0, v, jnp.exp(jnp.minimum(v, 0.0)) - 1.0)


def _mm_kernel(w_ref, x_ref, o_ref, *, act):
    y = jax.lax.dot_general(
        w_ref[...], x_ref[...], (((1,), (0,)), ((), ())),
        preferred_element_type=jnp.float32)
    o_ref[...] = _elu(y) if act else y


def _mm_act(w, xcols, act, nb=2048):
    co, k = w.shape
    k2, n = xcols.shape
    assert k == k2, (w.shape, xcols.shape)
    if n <= 4096:
        return pl.pallas_call(
            lambda wr, xr, orf: _mm_kernel(wr, xr, orf, act=act),
            out_shape=jax.ShapeDtypeStruct((co, n), jnp.float32),
        )(w, xcols)
    npad = (-n) % nb
    if npad:
        xcols = jnp.pad(xcols, ((0, 0), (0, npad)))
    ng = (n + npad) // nb
    out = pl.pallas_call(
        lambda wr, xr, orf: _mm_kernel(wr, xr, orf, act=act),
        grid=(ng,),
        in_specs=[
            pl.BlockSpec((co, k), lambda i: (0, 0)),
            pl.BlockSpec((k, nb), lambda i: (0, i)),
        ],
        out_specs=pl.BlockSpec((co, nb), lambda i: (0, i)),
        out_shape=jax.ShapeDtypeStruct((co, n + npad), jnp.float32),
    )(w, xcols)
    return out[:, :n] if npad else out


def _conv_layer(h, w, s, act):
    b, ci, t = h.shape
    co, ci2, k = w.shape
    to = -(-t // s)
    pad_total = max((to - 1) * s + k - t, 0)
    pad_lo = pad_total // 2
    xp = jnp.pad(h, ((0, 0), (0, 0), (pad_lo, pad_total - pad_lo)))
    cols = jnp.stack([xp[:, :, i::s][:, :, :to] for i in range(k)], axis=1)
    xf = cols.reshape(b, k * ci, to).transpose(1, 0, 2).reshape(k * ci, b * to)
    wf = w.transpose(0, 2, 1).reshape(co, k * ci)
    kk = ci * k
    if kk % 8:
        kp = (-kk) % 8
        wf = jnp.pad(wf, ((0, 0), (0, kp)))
        xf = jnp.pad(xf, ((0, kp), (0, 0)))
    cop = 0
    if co % 8:
        cop = (-co) % 8
        wf = jnp.pad(wf, ((0, cop), (0, 0)))
    y = _mm_act(wf, xf, False)
    if cop:
        y = y[:co]
    y = y.reshape(co, b, to).transpose(1, 0, 2)
    return jax.nn.elu(y) if act else y


def _upconv_layer(h, w, s, act):
    b, ci, m = h.shape
    co, ci2, k = w.shape
    pad_lo = (k + s - 2) // 2
    wparts = []
    for p in range(s):
        i0 = (pad_lo - p) % s
        e = -((pad_lo - p) // s)
        row = [jnp.zeros((co, ci), jnp.float32)] * 3
        for j in (0, 1):
            i = i0 + j * s
            o = e + j
            row = list(row)
            row[o + 1] = w[:, :, i]
        wparts.append(jnp.concatenate(row, axis=1))
    wbig = jnp.concatenate(wparts, axis=0)
    xp = jnp.pad(h, ((0, 0), (0, 0), (1, 1)))
    x3 = jnp.stack([xp[:, :, 0:m], xp[:, :, 1:m + 1], xp[:, :, 2:m + 2]],
                   axis=1).reshape(b, 3 * ci, m)
    xf = x3.transpose(1, 0, 2).reshape(3 * ci, b * m)
    y = _mm_act(wbig, xf, act)
    out = y.reshape(s, co, b, m).transpose(2, 1, 3, 0).reshape(b, co, m * s)
    return out


def _rvq_kernel(z_ref, cb_ref, cb2_ref, acc_ref, r_ref, d_ref):
    q = pl.program_id(0)

    @pl.when(q == 0)
    def _():
        r_ref[...] = z_ref[...]
        acc_ref[...] = jnp.zeros_like(acc_ref)

    r = r_ref[...]
    cb = cb_ref[0]
    cb2 = cb2_ref[0]
    r2 = jnp.sum(r * r, axis=1, keepdims=True)
    scores = jax.lax.dot_general(
        r, cb, (((1,), (1,)), ((), ())),
        preferred_element_type=jnp.float32)
    d_ref[...] = (r2 - 2.0 * scores) + cb2
    d = d_ref[...]
    dmin = jnp.min(d, axis=1, keepdims=True)
    iota = jax.lax.broadcasted_iota(jnp.int32, d.shape, 1)
    idx = jnp.min(jnp.where(d <= dmin, iota, d.shape[1]), axis=1,
                  keepdims=True)
    d_ref[...] = (iota == idx).astype(jnp.float32)
    quant = jax.lax.dot_general(
        d_ref[...], cb_ref[0], (((1,), (0,)), ((), ())),
        precision=jax.lax.Precision.HIGHEST,
        preferred_element_type=jnp.float32)
    r_ref[...] = r - quant
    acc_ref[...] = acc_ref[...] + quant


def _rvq(zrows, codebooks):
    r, d = zrows.shape
    nq, nc, d2 = codebooks.shape
    cb2 = jnp.sum(codebooks * codebooks, axis=2)[:, None, :]
    return pl.pallas_call(
        _rvq_kernel,
        grid=(nq,),
        in_specs=[
            pl.BlockSpec((r, d), lambda q: (0, 0)),
            pl.BlockSpec((1, nc, d2), lambda q: (q, 0, 0)),
            pl.BlockSpec((1, 1, nc), lambda q: (q, 0, 0)),
        ],
        out_specs=pl.BlockSpec((r, d), lambda q: (0, 0)),
        out_shape=jax.ShapeDtypeStruct((r, d), jnp.float32),
        scratch_shapes=[
            pltpu.VMEM((r, d), jnp.float32),
            pltpu.VMEM((r, nc), jnp.float32),
        ],
    )(zrows, codebooks, cb2)


def kernel(x, enc_w0, enc_w1, enc_w2, enc_w3, enc_w4, enc_wz, codebooks,
           dec_w0, dec_w1, dec_w2, dec_w3, dec_w4, dec_wo):
    b = x.shape[0]
    def _xconv(h, w, s):
        return lax.conv_general_dilated(h, w, (s,), 'SAME',
                                        dimension_numbers=('NCH', 'OIH', 'NCH'))
    h = jax.nn.elu(_xconv(x, enc_w0, 1))
    h = jax.nn.elu(_xconv(h, enc_w1, 2))
    h = jax.nn.elu(_xconv(h, enc_w2, 4))
    h = jax.nn.elu(_xconv(h, enc_w3, 5))
    h = jax.nn.elu(_xconv(h, enc_w4, 8))
    z = _xconv(h, enc_wz, 1)
    d = z.shape[1]
    t = z.shape[2]
    zrows = z.transpose(0, 2, 1).reshape(b * t, d)
    acc = _rvq(zrows, codebooks)
    zq = acc.reshape(b, t, d).transpose(0, 2, 1)
    g = _conv_layer(zq, dec_w0, 1, True)
    g = _upconv_layer(g, dec_w1, 8, True)
    g = _upconv_layer(g, dec_w2, 5, True)
    g = _upconv_layer(g, dec_w3, 4, True)
    g = _upconv_layer(g, dec_w4, 2, True)
    out = _conv_layer(g, dec_wo, 1, False)
    return out
